# Initial kernel scaffold; baseline (speedup 1.0000x reference)
import jax
import jax.numpy as jnp
from jax.experimental import pallas as pl


def kernel(x, edge_index, edge_attr, entity_embed, W1, b1, g1, be1, Wh, bh, gh, beh, Wq1, bq1, Wk1, bk1, Wv1, bv1, We1, Ws1, bs1, Wq2, bq2, Wk2, bk2, Wv2, bv2, We2, Ws2, bs2):
    raise NotImplementedError("write your pallas kernel here")


# speedup vs baseline: 44.0139x; 44.0139x over previous
import functools

import jax
import jax.numpy as jnp
from jax import lax
from jax.experimental import pallas as pl
from jax.experimental.pallas import tpu as pltpu
from jax.experimental.pallas import tpu_sc as plsc

N = 10000
E = 320000
INPUT_DIM = 128
NUM_EMB = 8
EMB = 16
EDGE_DIM = 4
EHID = 128
HID = 64
HEADS = 4
HC = HEADS * HID

_HIGH = lax.Precision.HIGHEST

C = 128
NCHUNK = E // C
NW = 32
NPT = N // 16
SROW = 80


def _ln(x, g, b, eps=1e-5):
    mu = jnp.mean(x, axis=-1, keepdims=True)
    xc = x - mu
    var = jnp.mean(xc * xc, axis=-1, keepdims=True)
    return xc * jax.lax.rsqrt(var + eps) * g + b


def _mlp_body(xg_ref, ea_ref, emb_ref, w1m_ref, b1_ref, g1_ref, be1_ref,
              whT_ref, bh_ref, gh_ref, beh_ref, o_ref):
    xg = xg_ref[...]
    ent = jnp.clip(xg_ref[:, 127:128].astype(jnp.int32), 0, NUM_EMB - 1)
    emb = jnp.zeros((xg.shape[0], EMB), jnp.float32)
    for k2 in range(NUM_EMB):
        emb = jnp.where(ent == k2, emb_ref[k2:k2 + 1, :], emb)
    msg = jnp.concatenate(
        [xg[:, :INPUT_DIM - 1], emb, ea_ref[...],
         jnp.zeros((xg.shape[0], 13), jnp.float32)], axis=1)
    mm = jnp.dot(msg.astype(jnp.bfloat16), w1m_ref[...],
                 preferred_element_type=jnp.float32)
    h = _ln(jnp.maximum(mm + b1_ref[...], 0.0), g1_ref[...], be1_ref[...])
    h2 = jnp.dot(h.astype(jnp.bfloat16), whT_ref[...],
                 preferred_element_type=jnp.float32) + bh_ref[...]
    o_ref[...] = _ln(jnp.maximum(h2, 0.0), gh_ref[...], beh_ref[...])


def _edge_mlp(xg, edge_attr, entity_embed, W1, b1, g1, be1, Wh, bh, gh, beh):
    W1m = jnp.concatenate(
        [W1.T, jnp.zeros((13, EHID), jnp.float32)], axis=0).astype(
            jnp.bfloat16)
    Be = 2000
    return pl.pallas_call(
        _mlp_body,
        grid=(E // Be,),
        in_specs=[
            pl.BlockSpec((Be, INPUT_DIM), lambda i: (i, 0)),
            pl.BlockSpec((Be, EDGE_DIM), lambda i: (i, 0)),
            pl.BlockSpec((NUM_EMB, EMB), lambda i: (0, 0)),
            pl.BlockSpec((160, EHID), lambda i: (0, 0)),
            pl.BlockSpec((1, EHID), lambda i: (0, 0)),
            pl.BlockSpec((1, EHID), lambda i: (0, 0)),
            pl.BlockSpec((1, EHID), lambda i: (0, 0)),
            pl.BlockSpec((EHID, EHID), lambda i: (0, 0)),
        ] + [pl.BlockSpec((1, EHID), lambda i: (0, 0))] * 3,
        out_specs=pl.BlockSpec((Be, EHID), lambda i: (i, 0)),
        out_shape=jax.ShapeDtypeStruct((E, EHID), jnp.float32),
    )(xg, edge_attr, entity_embed, W1m, b1[None, :], g1[None, :],
      be1[None, :], Wh.T.astype(jnp.bfloat16), bh[None, :], gh[None, :],
      beh[None, :])


def _sum_parts_body(p_ref, o_ref):
    o_ref[...] = jnp.sum(p_ref[...], axis=0)


def _sum_parts(parts):
    P, n, D = parts.shape
    Bn = 2000
    return pl.pallas_call(
        _sum_parts_body,
        grid=(n // Bn,),
        in_specs=[pl.BlockSpec((P, Bn, D), lambda i: (0, i, 0))],
        out_specs=pl.BlockSpec((Bn, D), lambda i: (i, 0)),
        out_shape=jax.ShapeDtypeStruct((n, D), jnp.float32),
    )(parts)


def _proj_body(x_ref, wqT_ref, bq_ref, wkT_ref, bk_ref, wvT_ref, bv_ref,
               wsT_ref, bs_ref, we_ref, q_ref, k_ref, v_ref, s_ref,
               qe_ref):
    xb = x_ref[...].astype(jnp.bfloat16)
    q = jnp.dot(xb, wqT_ref[...],
                preferred_element_type=jnp.float32) + bq_ref[...]
    q_ref[...] = q
    k_ref[...] = jnp.dot(xb, wkT_ref[...],
                         preferred_element_type=jnp.float32) + bk_ref[...]
    v_ref[...] = jnp.dot(xb, wvT_ref[...],
                         preferred_element_type=jnp.float32) + bv_ref[...]
    s_ref[...] = jnp.dot(xb, wsT_ref[...],
                         preferred_element_type=jnp.float32) + bs_ref[...]
    qes = []
    for h in range(HEADS):
        qes.append(jnp.dot(q[:, h * HID:(h + 1) * HID],
                           we_ref[h * HID:(h + 1) * HID, :],
                           precision=_HIGH))
    pad = jnp.zeros((q.shape[0], 128 - HEADS * EDGE_DIM), jnp.float32)
    qe_ref[...] = jnp.concatenate(qes + [pad], axis=1)


def _proj(X, Wq, bq, Wk, bk, Wv, bv, Ws, bs, We):
    Din = X.shape[1]
    Bn = 2000
    outs = pl.pallas_call(
        _proj_body,
        grid=(N // Bn,),
        in_specs=[
            pl.BlockSpec((Bn, Din), lambda i: (i, 0)),
            pl.BlockSpec((Din, HC), lambda i: (0, 0)),
            pl.BlockSpec((1, HC), lambda i: (0, 0)),
            pl.BlockSpec((Din, HC), lambda i: (0, 0)),
            pl.BlockSpec((1, HC), lambda i: (0, 0)),
            pl.BlockSpec((Din, HC), lambda i: (0, 0)),
            pl.BlockSpec((1, HC), lambda i: (0, 0)),
            pl.BlockSpec((Din, HC), lambda i: (0, 0)),
            pl.BlockSpec((1, HC), lambda i: (0, 0)),
            pl.BlockSpec((HC, EDGE_DIM), lambda i: (0, 0)),
        ],
        out_specs=[
            pl.BlockSpec((Bn, HC), lambda i: (i, 0)),
            pl.BlockSpec((Bn, HC), lambda i: (i, 0)),
            pl.BlockSpec((Bn, HC), lambda i: (i, 0)),
            pl.BlockSpec((Bn, HC), lambda i: (i, 0)),
            pl.BlockSpec((Bn, 128), lambda i: (i, 0)),
        ],
        out_shape=[
            jax.ShapeDtypeStruct((N, HC), jnp.float32),
            jax.ShapeDtypeStruct((N, HC), jnp.float32),
            jax.ShapeDtypeStruct((N, HC), jnp.float32),
            jax.ShapeDtypeStruct((N, HC), jnp.float32),
            jax.ShapeDtypeStruct((N, 128), jnp.float32),
        ],
    )(X, Wq.T.astype(jnp.bfloat16), bq[None, :],
      Wk.T.astype(jnp.bfloat16), bk[None, :],
      Wv.T.astype(jnp.bfloat16), bv[None, :],
      Ws.T.astype(jnp.bfloat16), bs[None, :],
      We.astype(jnp.bfloat16).astype(jnp.float32))
    Q, K, V, skip, QEw = outs
    QE = QEw[:, :HEADS * EDGE_DIM]
    return Q, K, V, skip, QE


def _m_merge_body(p_ref, o_ref):
    m = jnp.max(p_ref[...], axis=0)
    o_ref[...] = jnp.where(m == -jnp.inf, 0.0, m)


def _m_merge(parts):
    P = parts.shape[0]
    return pl.pallas_call(
        _m_merge_body,
        grid=(1,),
        in_specs=[pl.BlockSpec((P, HEADS, N), lambda i: (0, 0, 0))],
        out_specs=pl.BlockSpec((HEADS, N), lambda i: (0, 0)),
        out_shape=jax.ShapeDtypeStruct((HEADS, N), jnp.float32),
    )(parts)


def _combine_body(a_ref, skip_ref, weh_ref, o_ref):
    a = a_ref[...]
    outs = []
    for h in range(HEADS):
        c, l = divmod(h, 2)
        acc = a[c, l, :, 0:HID]
        s = a[c, l, :, HID:HID + 1] + 1e-16
        for j in range(EDGE_DIM):
            t = a[c, l, :, HID + 1 + j:HID + 2 + j]
            acc += t * weh_ref[h, j:j + 1, :]
        outs.append(acc / s)
    o_ref[...] = jnp.concatenate(outs, axis=1) + skip_ref[...]


def _combine(agg_parts, skip, We):
    weh = We.astype(jnp.bfloat16).astype(jnp.float32).reshape(
        HEADS, HID, EDGE_DIM).transpose(0, 2, 1)
    Bn = 1000
    return pl.pallas_call(
        _combine_body,
        grid=(N // Bn,),
        in_specs=[
            pl.BlockSpec((2, 2, Bn, 128), lambda i: (0, 0, i, 0)),
            pl.BlockSpec((Bn, HC), lambda i: (i, 0)),
            pl.BlockSpec((HEADS, EDGE_DIM, HID), lambda i: (0, 0, 0)),
        ],
        out_specs=pl.BlockSpec((Bn, HC), lambda i: (i, 0)),
        out_shape=jax.ShapeDtypeStruct((N, HC), jnp.float32),
    )(agg_parts, skip, weh)


_MESH = None


def _mesh():
    global _MESH
    if _MESH is None:
        _MESH = plsc.VectorSubcoreMesh(core_axis_name="c",
                                       subcore_axis_name="s")
    return _MESH


def _rbf16(v):
    u = plsc.bitcast(v, jnp.uint32)
    u = (u + jnp.uint32(0x7FFF) + ((u >> 16) & jnp.uint32(1))) \
        & jnp.uint32(0xFFFF0000)
    return plsc.bitcast(u, jnp.float32)


def _wid():
    return lax.axis_index("s") * 2 + lax.axis_index("c")


def _sc_gather(node_part, src):
    @functools.partial(
        pl.kernel, mesh=_mesh(),
        compiler_params=pltpu.CompilerParams(use_tc_tiling_on_sc=False, needs_layout_passes=False),
        out_type=jax.ShapeDtypeStruct((E, EHID), jnp.float32),
        scratch_types=[
            pltpu.VMEM((C,), jnp.int32),
            pltpu.VMEM((C, EHID), jnp.float32),
            pltpu.SemaphoreType.DMA,
        ],
    )
    def k(np_hbm, src_hbm, out_hbm, idx_v, rows_v, sem):
        w = _wid()

        def body(i, carry):
            ci = w + NW * i

            @pl.when(ci < NCHUNK)
            def _():
                base = ci * C
                pltpu.sync_copy(src_hbm.at[pl.ds(base, C)], idx_v)
                pltpu.async_copy(np_hbm.at[idx_v], rows_v, sem).wait()
                pltpu.sync_copy(rows_v, out_hbm.at[pl.ds(base, C)])
            return carry

        lax.fori_loop(0, (NCHUNK + NW - 1) // NW, body, 0)

    return k(node_part, src)


def _sc_scatter_add(h2e, dst):
    @functools.partial(
        pl.kernel, mesh=_mesh(),
        compiler_params=pltpu.CompilerParams(use_tc_tiling_on_sc=False, needs_layout_passes=False),
        out_type=jax.ShapeDtypeStruct((2, N, EHID), jnp.float32),
        scratch_types=[
            pltpu.VMEM((C,), jnp.int32),
            pltpu.VMEM((C, EHID), jnp.float32),
            pltpu.VMEM((NPT // 5, EHID), jnp.float32),
            pltpu.VMEM_SHARED((N, EHID), jnp.float32),
            pltpu.SemaphoreType.DMA,
        ],
    )
    def k(rows_hbm, dst_hbm, out_hbm, idx_v, rows_v, zbuf, acc_sh, sem):
        cid = lax.axis_index("c")
        sid = lax.axis_index("s")
        w = sid * 2 + cid
        r0 = sid * NPT

        def zb2(i, carry):
            def zrow(j, c2):
                zbuf[i, pl.ds(j * 16, 16)] = jnp.zeros((16,), jnp.float32)
                return c2
            return lax.fori_loop(0, EHID // 16, zrow, carry)

        lax.fori_loop(0, NPT // 5, zb2, 0)
        for kk in range(5):
            pltpu.sync_copy(zbuf,
                            acc_sh.at[pl.ds(r0 + kk * (NPT // 5), NPT // 5)])
        plsc.subcore_barrier()

        def body(i, carry):
            ci = w + NW * i

            @pl.when(ci < NCHUNK)
            def _():
                base = ci * C
                pltpu.sync_copy(dst_hbm.at[pl.ds(base, C)], idx_v)
                pltpu.sync_copy(rows_hbm.at[pl.ds(base, C)], rows_v)
                pltpu.sync_copy(rows_v, acc_sh.at[idx_v], add=True)
            return carry

        lax.fori_loop(0, (NCHUNK + NW - 1) // NW, body, 0)
        plsc.subcore_barrier()
        f0 = pl.multiple_of(sid * 624, 8)
        pltpu.sync_copy(acc_sh.at[pl.ds(f0, 624)],
                        out_hbm.at[cid, pl.ds(f0, 624)])

        @pl.when(sid == 0)
        def _():
            pltpu.sync_copy(acc_sh.at[pl.ds(9984, 16)],
                            out_hbm.at[cid, pl.ds(9984, 16)])

    return k(h2e, dst)


def _sc_logits(Q, K, QE, src, dst, edge_attr):
    isq = 0.125

    @functools.partial(
        pl.kernel, mesh=_mesh(),
        compiler_params=pltpu.CompilerParams(use_tc_tiling_on_sc=False, needs_layout_passes=False),
        out_type=(jax.ShapeDtypeStruct((HEADS, E), jnp.float32),
                  jax.ShapeDtypeStruct((NW, HEADS, N), jnp.float32)),
        scratch_types=[
            pltpu.VMEM((C,), jnp.int32),
            pltpu.VMEM((C,), jnp.int32),
            pltpu.VMEM((C, HC), jnp.float32),
            pltpu.VMEM((C, HC), jnp.float32),
            pltpu.VMEM((C, HEADS * EDGE_DIM), jnp.float32),
            pltpu.VMEM((C, EDGE_DIM), jnp.float32),
            pltpu.VMEM((HEADS, C), jnp.float32),
            pltpu.VMEM((HEADS, N), jnp.float32),
            pltpu.VMEM((16,), jnp.int32),
            pltpu.VMEM((16,), jnp.float32),
            pltpu.SemaphoreType.DMA,
        ],
    )
    def k(q_hbm, k_hbm, qe_hbm, src_hbm, dst_hbm, ea_hbm, lg_hbm, m_hbm,
          sidx, didx, qrows, krows, qerows, earows, lstage, mtile,
          kstage, vstage, sem):
        w = _wid()
        iota16 = jnp.arange(16, dtype=jnp.int32)

        def seg_max_update(dv, lvec, mref):
            sk, sv = plsc.sort_key_val(dv, lvec)
            kstage[pl.ds(0, 16)] = sk
            for sh in (1, 2, 4, 8):
                idx = jnp.maximum(iota16 - sh, 0)
                kg = plsc.load_gather(kstage, [idx])
                vstage[pl.ds(0, 16)] = sv
                vg = plsc.load_gather(vstage, [idx])
                sv = jnp.where(kg == sk, jnp.maximum(sv, vg), sv)
            kn = plsc.load_gather(kstage, [jnp.minimum(iota16 + 1, 15)])
            mask = (sk != kn) | (iota16 == 15)
            mg = plsc.load_gather(mref, [sk])
            plsc.store_scatter(mref, [sk], jnp.maximum(mg, sv), mask=mask)

        def minit_h(h, carry):
            def row(i, c2):
                mtile[h, pl.ds(i * 16, 16)] = jnp.full((16,), -jnp.inf,
                                                       jnp.float32)
                return c2
            return lax.fori_loop(0, N // 16, row, carry)

        lax.fori_loop(0, HEADS, minit_h, 0)

        def body(i, carry):
            ci = w + NW * i

            @pl.when(ci < NCHUNK)
            def _():
                base = ci * C
                pltpu.sync_copy(src_hbm.at[pl.ds(base, C)], sidx)
                pltpu.sync_copy(dst_hbm.at[pl.ds(base, C)], didx)
                pltpu.sync_copy(ea_hbm.at[pl.ds(base, C)], earows)
                cp1 = pltpu.async_copy(q_hbm.at[didx], qrows, sem)
                cp2 = pltpu.async_copy(k_hbm.at[sidx], krows, sem)
                cp3 = pltpu.async_copy(qe_hbm.at[didx], qerows, sem)
                cp1.wait()
                cp2.wait()
                cp3.wait()

                def grp(g, c2):
                    rows = jnp.arange(16, dtype=jnp.int32) + g * 16
                    dv = didx[pl.ds(g * 16, 16)]

                    def dbody(d, accs):
                        new = []
                        for h in range(HEADS):
                            col = jnp.full((16,), h * HID, jnp.int32) + d
                            qv = plsc.load_gather(qrows, [rows, col])
                            kv = plsc.load_gather(krows, [rows, col])
                            new.append(accs[h] + qv * kv)
                        return tuple(new)

                    accs = lax.fori_loop(
                        0, HID, dbody,
                        tuple(jnp.zeros((16,), jnp.float32)
                              for _ in range(HEADS)))
                    eav = [_rbf16(plsc.load_gather(
                        earows, [rows, jnp.full((16,), j, jnp.int32)]))
                        for j in range(EDGE_DIM)]
                    for h in range(HEADS):
                        qet = jnp.zeros((16,), jnp.float32)
                        for j in range(EDGE_DIM):
                            qev = plsc.load_gather(
                                qerows,
                                [rows,
                                 jnp.full((16,), h * EDGE_DIM + j,
                                          jnp.int32)])
                            qet += qev * eav[j]
                        lvec = (accs[h] + qet) * isq
                        lstage[h, pl.ds(g * 16, 16)] = lvec
                        seg_max_update(dv, lvec, mtile.at[h])
                    return c2

                lax.fori_loop(0, C // 16, grp, 0)
                for h in range(HEADS):
                    pltpu.sync_copy(lstage.at[h],
                                    lg_hbm.at[h, pl.ds(base, C)])
            return carry

        lax.fori_loop(0, (NCHUNK + NW - 1) // NW, body, 0)
        pltpu.sync_copy(mtile, m_hbm.at[w])

    return k(Q, K, QE, src, dst, edge_attr)


def _sc_agg(logits, m, V4, src, dst, edge_attr):
    NCS = 16

    @functools.partial(
        pl.kernel, mesh=_mesh(),
        compiler_params=pltpu.CompilerParams(use_tc_tiling_on_sc=False, needs_layout_passes=False),
        out_type=jax.ShapeDtypeStruct((2, 2, N, 128), jnp.float32),
        scratch_types=[
            pltpu.VMEM((C,), jnp.int32),
            pltpu.VMEM((C,), jnp.int32),
            pltpu.VMEM((C,), jnp.int32),
            pltpu.VMEM((C, HID), jnp.float32),
            pltpu.VMEM((C, EDGE_DIM), jnp.float32),
            pltpu.VMEM((C,), jnp.float32),
            pltpu.VMEM((N,), jnp.float32),
            pltpu.VMEM((C, SROW), jnp.float32),
            pltpu.VMEM((NPT // 5, SROW), jnp.float32),
            pltpu.VMEM_SHARED((N, SROW), jnp.float32),
            pltpu.SemaphoreType.DMA,
        ],
    )
    def k(lg_hbm, m_hbm, v_hbm, src_hbm, dst_hbm, ea_hbm, out_hbm,
          sidx, didx, vidx, vrows, earows, lrow, mrow, wstage,
          zbuf, acc_sh, sem):
        cid = lax.axis_index("c")
        sid = lax.axis_index("s")
        r0 = sid * NPT
        iota = jnp.arange(16, dtype=jnp.int32)

        def zb2(i, carry):
            def zrow(j, c2):
                zbuf[i, pl.ds(j * 16, 16)] = jnp.zeros((16,), jnp.float32)
                return c2
            return lax.fori_loop(0, SROW // 16, zrow, carry)

        lax.fori_loop(0, NPT // 5, zb2, 0)

        def zw(i, carry):
            wstage[i, pl.ds(SROW - 16, 16)] = jnp.zeros((16,), jnp.float32)
            return carry

        lax.fori_loop(0, C, zw, 0)

        for l in range(2):
            for kk in range(5):
                pltpu.sync_copy(
                    zbuf, acc_sh.at[pl.ds(r0 + kk * (NPT // 5), NPT // 5)])
            pltpu.sync_copy(m_hbm.at[2 * cid + l], mrow)
            plsc.subcore_barrier()

            def body(i, carry):
                ci = sid + NCS * i

                @pl.when(ci < NCHUNK)
                def _():
                    base = ci * C
                    pltpu.sync_copy(src_hbm.at[pl.ds(base, C)], sidx)
                    pltpu.sync_copy(dst_hbm.at[pl.ds(base, C)], didx)
                    pltpu.sync_copy(ea_hbm.at[pl.ds(base, C)], earows)
                    pltpu.sync_copy(lg_hbm.at[2 * cid + l, pl.ds(base, C)],
                                    lrow)

                    def vb(g, c2):
                        sv = sidx[pl.ds(g * 16, 16)]
                        vidx[pl.ds(g * 16, 16)] = sv * 4 + (2 * cid + l)
                        return c2

                    lax.fori_loop(0, C // 16, vb, 0)
                    cp = pltpu.async_copy(v_hbm.at[vidx], vrows, sem)
                    cp.wait()

                    def grp(g, c2):
                        rows = iota + g * 16
                        dv = didx[pl.ds(g * 16, 16)]
                        mg = plsc.load_gather(mrow, [dv])
                        lv = lrow[pl.ds(g * 16, 16)]
                        exv = jnp.exp(lv - mg)
                        plsc.store_scatter(
                            wstage, [rows, jnp.full((16,), HID, jnp.int32)],
                            exv)
                        for j in range(EDGE_DIM):
                            eaj = plsc.load_gather(
                                earows,
                                [rows, jnp.full((16,), j, jnp.int32)])
                            plsc.store_scatter(
                                wstage,
                                [rows,
                                 jnp.full((16,), HID + 1 + j, jnp.int32)],
                                exv * _rbf16(eaj))
                        for i2 in range(16):
                            e = g * 16 + i2
                            exi = jnp.sum(
                                jnp.where(iota == i2, exv, 0.0), axis=0)
                            for r in range(HID // 16):
                                wstage[e, pl.ds(r * 16, 16)] = (
                                    vrows[e, pl.ds(r * 16, 16)] * exi)
                        return c2

                    lax.fori_loop(0, C // 16, grp, 0)
                    pltpu.sync_copy(wstage, acc_sh.at[didx], add=True)
                return carry

            lax.fori_loop(0, (NCHUNK + NCS - 1) // NCS, body, 0)
            plsc.subcore_barrier()
            f0 = pl.multiple_of(sid * 624, 8)
            pltpu.sync_copy(acc_sh.at[pl.ds(f0, 624)],
                            out_hbm.at[cid, l, pl.ds(f0, 624),
                                       pl.ds(0, SROW)])

            @pl.when(sid == 0)
            def _():
                pltpu.sync_copy(acc_sh.at[pl.ds(9984, 16)],
                                out_hbm.at[cid, l, pl.ds(9984, 16),
                                           pl.ds(0, SROW)])
            plsc.subcore_barrier()

    return k(logits, m, V4, src, dst, edge_attr)


def _tconv_layer(X, src, dst, edge_attr, Wq, bq, Wk, bk, Wv, bv, We, Ws, bs):
    Q, K, V, skip, QE = _proj(X, Wq, bq, Wk, bk, Wv, bv, Ws, bs, We)
    logits, m_parts = _sc_logits(Q, K, QE, src, dst, edge_attr)
    m = _m_merge(m_parts)
    V4 = V.reshape(N * HEADS, HID)
    agg_parts = _sc_agg(logits, m, V4, src, dst, edge_attr)
    return _combine(agg_parts, skip, We)


def kernel(x, edge_index, edge_attr, entity_embed, W1, b1, g1, be1, Wh, bh,
           gh, beh, Wq1, bq1, Wk1, bk1, Wv1, bv1, We1, Ws1, bs1,
           Wq2, bq2, Wk2, bk2, Wv2, bv2, We2, Ws2, bs2):
    src = edge_index[0]
    dst = edge_index[1]
    gathered = _sc_gather(x, src)
    h2e = _edge_mlp(gathered, edge_attr, entity_embed, W1, b1, g1, be1,
                    Wh, bh, gh, beh)
    node = _sum_parts(_sc_scatter_add(h2e, dst))
    h1 = _tconv_layer(node, src, dst, edge_attr, Wq1, bq1, Wk1, bk1, Wv1,
                      bv1, We1, Ws1, bs1)
    h2 = _tconv_layer(h1, src, dst, edge_attr, Wq2, bq2, Wk2, bk2, Wv2,
                      bv2, We2, Ws2, bs2)
    return h2
